# Initial kernel scaffold; baseline (speedup 1.0000x reference)
import jax
import jax.numpy as jnp
from jax.experimental import pallas as pl


def kernel(x, edge_index, batch, W1, b1, Wc1, bc1, Wc2, bc2, W2, b2):
    raise NotImplementedError("write your pallas kernel here")


# speedup vs baseline: 16.7057x; 16.7057x over previous
import dataclasses
import functools

import jax
import jax.numpy as jnp
from jax import lax
from jax.experimental import pallas as pl
from jax.experimental.pallas import tpu as pltpu
from jax.experimental.pallas import tpu_sc as plsc

NC = 2
NS = 16
LN = 16
CH = 128
TRASH = 8


def _vector_mesh():
    return plsc.VectorSubcoreMesh(core_axis_name="c", subcore_axis_name="s")


def _sc_compiler_params():
    cp = pltpu.CompilerParams()
    if "needs_layout_passes" in pltpu.CompilerParams.__dataclass_fields__:
        cp = dataclasses.replace(cp, needs_layout_passes=False)
    return cp


def _sc_degree(dst, n_nodes):
    e = dst.shape[0]
    nw = NC * NS
    epw = e // nw
    assert epw * nw == e

    @functools.partial(
        pl.kernel,
        mesh=_vector_mesh(),
        out_type=jax.ShapeDtypeStruct((nw * n_nodes,), jnp.float32),
        scratch_types=[
            pltpu.VMEM((epw,), jnp.int32),
            pltpu.VMEM((n_nodes,), jnp.float32),
        ],
        compiler_params=_sc_compiler_params(),
    )
    def k(dst_hbm, out_hbm, dbuf, hist):
        w = lax.axis_index("s") * NC + lax.axis_index("c")
        pltpu.sync_copy(dst_hbm.at[pl.ds(w * epw, epw)], dbuf)

        zeros = jnp.zeros((LN,), jnp.float32)

        @pl.loop(0, n_nodes, step=LN)
        def _(i):
            hist[pl.ds(i, LN)] = zeros

        ones = jnp.ones((LN,), jnp.float32)

        @pl.loop(0, epw, step=LN)
        def _(i):
            plsc.addupdate_scatter(hist, [dbuf[pl.ds(i, LN)]], ones)

        pltpu.sync_copy(hist, out_hbm.at[pl.ds(w * n_nodes, n_nodes)])

    return k(dst)


def _sc_scatter(g, idx, n_nodes, fh):
    nb = idx.shape[2]
    rps0 = (-(-n_nodes // NS) + 7) // 8 * 8
    rps_last = n_nodes - rps0 * (NS - 1)
    assert 0 < rps_last <= rps0
    assert nb % 4 == 0

    @functools.partial(
        pl.kernel,
        mesh=_vector_mesh(),
        out_type=jax.ShapeDtypeStruct((NC, n_nodes, fh), jnp.float32),
        scratch_types=[
            pltpu.VMEM((4, 2, CH), jnp.int32),
            pltpu.VMEM((2, CH, fh), jnp.float32),
            pltpu.VMEM_SHARED((n_nodes + TRASH, fh), jnp.float32),
            pltpu.SemaphoreType.DMA,
            pltpu.SemaphoreType.DMA,
            pltpu.SemaphoreType.DMA,
            pltpu.SemaphoreType.DMA,
            pltpu.SemaphoreType.DMA,
            pltpu.SemaphoreType.DMA,
            pltpu.SemaphoreType.DMA,
            pltpu.SemaphoreType.DMA,
        ],
    )
    def k(g_hbm, idx_hbm, out_hbm, ibuf, rows, acc,
          si0, si1, si2, si3, sg0, sg1, ss0, ss1):
        c = lax.axis_index("c")
        s = lax.axis_index("s")
        base = s * rps0

        @pl.when(s < NS - 1)
        def _():
            pltpu.sync_copy(
                g_hbm.at[pl.ds(c * n_nodes + base, rps0)],
                acc.at[pl.ds(base, rps0)],
            )

        @pl.when(s == NS - 1)
        def _():
            pltpu.sync_copy(
                g_hbm.at[pl.ds(c * n_nodes + base, rps_last)],
                acc.at[pl.ds(base, rps_last)],
            )
        plsc.subcore_barrier()

        sis = (si0, si1, si2, si3)
        sgs = (sg0, sg1)
        widx = idx_hbm.at[c].at[s]

        def start_idx(j, r):
            pltpu.async_copy(widx.at[j], ibuf.at[r], sis[r])

        def wait_idx(j, r):
            pltpu.make_async_copy(widx.at[j], ibuf.at[r], sis[r]).wait()

        def start_g(r, b):
            pltpu.async_copy(g_hbm.at[ibuf.at[r, 0]], rows.at[b], sgs[b])

        def wait_g(r, b):
            pltpu.make_async_copy(g_hbm.at[ibuf.at[r, 0]], rows.at[b], sgs[b]).wait()

        def scat(r, b):
            pltpu.sync_copy(rows.at[b], acc.at[ibuf.at[r, 1]], add=True)

        start_idx(0, 0)
        start_idx(1, 1)
        start_idx(2, 2)
        start_idx(3, 3)
        wait_idx(0, 0)
        start_g(0, 0)

        @pl.loop(0, nb // 4)
        def _(i):
            j = 4 * i
            for t in range(4):
                rj = t
                rn = (t + 1) % 4
                b = t % 2
                k_ = j + t

                @pl.when(k_ + 1 < nb)
                def _():
                    wait_idx(k_ + 1, rn)
                    start_g(rn, 1 - b)

                wait_g(rj, b)
                scat(rj, b)

                @pl.when(k_ + 4 < nb)
                def _():
                    start_idx(k_ + 4, rj)

        plsc.subcore_barrier()

        @pl.when(s < NS - 1)
        def _():
            pltpu.sync_copy(
                acc.at[pl.ds(base, rps0)],
                out_hbm.at[c].at[pl.ds(base, rps0)],
            )

        @pl.when(s == NS - 1)
        def _():
            pltpu.sync_copy(
                acc.at[pl.ds(base, rps_last)],
                out_hbm.at[c].at[pl.ds(base, rps_last)],
            )

    return k(g, idx)


_MT = 1000


def _tc_mlp1(x, W1, b1, Wc1):
    n, f = x.shape
    h = W1.shape[1]

    def body(x_ref, w1_ref, b1_ref, wc1_ref, out_ref):
        h1 = jnp.maximum(
            jnp.dot(x_ref[...], w1_ref[...], preferred_element_type=jnp.float32, precision=lax.Precision.HIGHEST)
            + b1_ref[...],
            0.0,
        )
        out_ref[...] = jnp.dot(h1, wc1_ref[...], preferred_element_type=jnp.float32, precision=lax.Precision.HIGHEST)

    return pl.pallas_call(
        body,
        grid=(n // _MT,),
        in_specs=[
            pl.BlockSpec((_MT, f), lambda i: (i, 0)),
            pl.BlockSpec((f, h), lambda i: (0, 0)),
            pl.BlockSpec((1, h), lambda i: (0, 0)),
            pl.BlockSpec((h, h), lambda i: (0, 0)),
        ],
        out_specs=pl.BlockSpec((_MT, h), lambda i: (i, 0)),
        out_shape=jax.ShapeDtypeStruct((n, h), jnp.float32),
    )(x, W1, b1.reshape(1, -1), Wc1)


def _tc_scale_split(partials, hw, fh):
    nw, n, _ = partials.shape
    h = hw.shape[1]

    def body(p_ref, hw_ref, g_ref, dis_ref):
        deg = jnp.sum(p_ref[...], axis=0) + 1.0
        dis = lax.rsqrt(deg)
        dis_ref[...] = dis
        g = hw_ref[...] * dis
        g_ref[0, :, :] = g[:, :fh]
        g_ref[1, :, :] = g[:, fh:]

    return pl.pallas_call(
        body,
        grid=(n // _MT,),
        in_specs=[
            pl.BlockSpec((nw, _MT, 1), lambda i: (0, i, 0)),
            pl.BlockSpec((_MT, h), lambda i: (i, 0)),
        ],
        out_specs=[
            pl.BlockSpec((NC, _MT, fh), lambda i: (0, i, 0)),
            pl.BlockSpec((_MT, 1), lambda i: (i, 0)),
        ],
        out_shape=[
            jax.ShapeDtypeStruct((NC, n, fh), jnp.float32),
            jax.ShapeDtypeStruct((n, 1), jnp.float32),
        ],
    )(partials, hw)


def _tc_mid(acc, dis, bc, Wc, fh):
    n = dis.shape[0]
    h = Wc.shape[0]

    def body(acc_ref, dis_ref, b_ref, w_ref, g_ref):
        a = jnp.concatenate([acc_ref[0], acc_ref[1]], axis=1)
        d = dis_ref[...]
        out1 = a * d + b_ref[...]
        h2 = jnp.where(out1 > 0, out1, 0.01 * out1)
        hw = jnp.dot(h2, w_ref[...], preferred_element_type=jnp.float32, precision=lax.Precision.HIGHEST)
        g = hw * d
        g_ref[0, :, :] = g[:, :fh]
        g_ref[1, :, :] = g[:, fh:]

    return pl.pallas_call(
        body,
        grid=(n // _MT,),
        in_specs=[
            pl.BlockSpec((NC, _MT, fh), lambda i: (0, i, 0)),
            pl.BlockSpec((_MT, 1), lambda i: (i, 0)),
            pl.BlockSpec((1, h), lambda i: (0, 0)),
            pl.BlockSpec((h, h), lambda i: (0, 0)),
        ],
        out_specs=pl.BlockSpec((NC, _MT, fh), lambda i: (0, i, 0)),
        out_shape=jax.ShapeDtypeStruct((NC, n, fh), jnp.float32),
    )(acc, dis, bc.reshape(1, -1), Wc)


def _tc_final(acc, dis, bc, W2, b2, batch2d, n_graphs, fh):
    n = dis.shape[0]
    h = W2.shape[0]

    def body(acc_ref, dis_ref, bc_ref, w_ref, b2_ref, batch_ref, emb_ref, h4_ref):
        i = pl.program_id(0)
        a = jnp.concatenate([acc_ref[0], acc_ref[1]], axis=1)
        out2 = a * dis_ref[...] + bc_ref[...]
        h3 = jnp.where(out2 > 0, out2, 0.01 * out2)
        h4 = jnp.dot(h3, w_ref[...], preferred_element_type=jnp.float32, precision=lax.Precision.HIGHEST) + b2_ref[...]
        h4_ref[...] = h4
        seg = batch_ref[...]
        ids = lax.broadcasted_iota(jnp.int32, (_MT, n_graphs), 1)
        oh = (seg == ids).astype(jnp.float32)
        part = lax.dot_general(
            oh, h4, (((0,), (0,)), ((), ())),
            preferred_element_type=jnp.float32, precision=lax.Precision.HIGHEST,
        )

        @pl.when(i == 0)
        def _():
            emb_ref[...] = part

        @pl.when(i > 0)
        def _():
            emb_ref[...] += part

    return pl.pallas_call(
        body,
        grid=(n // _MT,),
        in_specs=[
            pl.BlockSpec((NC, _MT, fh), lambda i: (0, i, 0)),
            pl.BlockSpec((_MT, 1), lambda i: (i, 0)),
            pl.BlockSpec((1, h), lambda i: (0, 0)),
            pl.BlockSpec((h, h), lambda i: (0, 0)),
            pl.BlockSpec((1, h), lambda i: (0, 0)),
            pl.BlockSpec((_MT, 1), lambda i: (i, 0)),
        ],
        out_specs=[
            pl.BlockSpec((n_graphs, h), lambda i: (0, 0)),
            pl.BlockSpec((_MT, h), lambda i: (i, 0)),
        ],
        out_shape=[
            jax.ShapeDtypeStruct((n_graphs, h), jnp.float32),
            jax.ShapeDtypeStruct((n, h), jnp.float32),
        ],
    )(acc, dis, bc.reshape(1, -1), W2, b2.reshape(1, -1), batch2d)


def kernel(x, edge_index, batch, W1, b1, Wc1, bc1, Wc2, bc2, W2, b2):
    n, _ = x.shape
    h = W1.shape[1]
    fh = h // NC
    e = edge_index.shape[1]
    g_count = 64

    src = edge_index[0]
    dst = edge_index[1]

    eps_real = e // NS
    nb = (-(-eps_real // CH) + 3) // 4 * 4
    epw = nb * CH
    padk = epw - eps_real
    src_r = src.reshape(NS, eps_real)
    dst_r = dst.reshape(NS, eps_real)
    if padk:
        pad_src = (
            jnp.arange(padk, dtype=jnp.int32)[None, :] * 131
            + jnp.arange(NS, dtype=jnp.int32)[:, None] * 977
        ) % n
        pad_dst = jnp.broadcast_to(
            n + (jnp.arange(padk, dtype=jnp.int32) % TRASH)[None, :], (NS, padk)
        )
        src_p = jnp.concatenate([src_r, pad_src], axis=1)
        dst_p = jnp.concatenate([dst_r, pad_dst], axis=1)
    else:
        src_p, dst_p = src_r, dst_r
    dst_c = dst_p.reshape(NS, nb, 1, CH)
    idx = jnp.stack(
        [
            jnp.concatenate([src_p.reshape(NS, nb, 1, CH), dst_c], axis=2),
            jnp.concatenate([(src_p + n).reshape(NS, nb, 1, CH), dst_c], axis=2),
        ]
    )

    partials = _sc_degree(dst, n).reshape(NC * NS, n, 1)
    hw1 = _tc_mlp1(x, W1, b1, Wc1)
    g1, dis = _tc_scale_split(partials, hw1, fh)
    acc1 = _sc_scatter(g1.reshape(NC * n, fh), idx, n, fh)
    g2 = _tc_mid(acc1, dis, bc1, Wc2, fh)
    acc2 = _sc_scatter(g2.reshape(NC * n, fh), idx, n, fh)
    emb, h4 = _tc_final(acc2, dis, bc2, W2, b2, batch.reshape(n, 1), g_count, fh)
    return emb, h4


# speedup vs baseline: 17.6158x; 1.0545x over previous
import dataclasses
import functools

import jax
import jax.numpy as jnp
from jax import lax
from jax.experimental import pallas as pl
from jax.experimental.pallas import tpu as pltpu
from jax.experimental.pallas import tpu_sc as plsc

NC = 2
NS = 16
LN = 16
CH = 128
TRASH = 8


def _vector_mesh():
    return plsc.VectorSubcoreMesh(core_axis_name="c", subcore_axis_name="s")


def _sc_compiler_params():
    cp = pltpu.CompilerParams()
    if "needs_layout_passes" in pltpu.CompilerParams.__dataclass_fields__:
        cp = dataclasses.replace(cp, needs_layout_passes=False)
    return cp


def _sc_degree(dst, n_nodes):
    e = dst.shape[0]
    nw = NC * NS
    epw = e // nw
    assert epw * nw == e

    @functools.partial(
        pl.kernel,
        mesh=_vector_mesh(),
        out_type=jax.ShapeDtypeStruct((nw * n_nodes,), jnp.float32),
        scratch_types=[
            pltpu.VMEM((epw,), jnp.int32),
            pltpu.VMEM((n_nodes,), jnp.float32),
        ],
        compiler_params=_sc_compiler_params(),
    )
    def k(dst_hbm, out_hbm, dbuf, hist):
        w = lax.axis_index("s") * NC + lax.axis_index("c")
        pltpu.sync_copy(dst_hbm.at[pl.ds(w * epw, epw)], dbuf)

        zeros = jnp.zeros((LN,), jnp.float32)

        @pl.loop(0, n_nodes, step=LN)
        def _(i):
            hist[pl.ds(i, LN)] = zeros

        ones = jnp.ones((LN,), jnp.float32)

        @pl.loop(0, epw, step=LN)
        def _(i):
            plsc.addupdate_scatter(hist, [dbuf[pl.ds(i, LN)]], ones)

        pltpu.sync_copy(hist, out_hbm.at[pl.ds(w * n_nodes, n_nodes)])

    return k(dst)


def _sc_scatter(g, idx, n_nodes, fh):
    nb = idx.shape[2]
    rps0 = (-(-n_nodes // NS) + 7) // 8 * 8
    rps_last = n_nodes - rps0 * (NS - 1)
    assert 0 < rps_last <= rps0
    assert nb % 4 == 0

    @functools.partial(
        pl.kernel,
        mesh=_vector_mesh(),
        out_type=jax.ShapeDtypeStruct((NC, n_nodes, fh), jnp.float32),
        scratch_types=[
            pltpu.VMEM((4, 2, CH), jnp.int32),
            pltpu.VMEM((2, CH, fh), jnp.float32),
            pltpu.VMEM_SHARED((n_nodes + TRASH, fh), jnp.float32),
            pltpu.SemaphoreType.DMA,
            pltpu.SemaphoreType.DMA,
            pltpu.SemaphoreType.DMA,
            pltpu.SemaphoreType.DMA,
            pltpu.SemaphoreType.DMA,
            pltpu.SemaphoreType.DMA,
            pltpu.SemaphoreType.DMA,
            pltpu.SemaphoreType.DMA,
        ],
    )
    def k(g_hbm, idx_hbm, out_hbm, ibuf, rows, acc,
          si0, si1, si2, si3, sg0, sg1, ss0, ss1):
        c = lax.axis_index("c")
        s = lax.axis_index("s")
        base = s * rps0

        @pl.when(s < NS - 1)
        def _():
            pltpu.sync_copy(
                g_hbm.at[pl.ds(c * n_nodes + base, rps0)],
                acc.at[pl.ds(base, rps0)],
            )

        @pl.when(s == NS - 1)
        def _():
            pltpu.sync_copy(
                g_hbm.at[pl.ds(c * n_nodes + base, rps_last)],
                acc.at[pl.ds(base, rps_last)],
            )
        plsc.subcore_barrier()

        sis = (si0, si1, si2, si3)
        sgs = (sg0, sg1)
        widx = idx_hbm.at[c].at[s]

        def start_idx(j, r):
            pltpu.async_copy(widx.at[j], ibuf.at[r], sis[r])

        def wait_idx(j, r):
            pltpu.make_async_copy(widx.at[j], ibuf.at[r], sis[r]).wait()

        def start_g(r, b):
            pltpu.async_copy(g_hbm.at[ibuf.at[r, 0]], rows.at[b], sgs[b])

        def wait_g(r, b):
            pltpu.make_async_copy(g_hbm.at[ibuf.at[r, 0]], rows.at[b], sgs[b]).wait()

        def scat(r, b):
            pltpu.sync_copy(rows.at[b], acc.at[ibuf.at[r, 1]], add=True)

        start_idx(0, 0)
        start_idx(1, 1)
        start_idx(2, 2)
        start_idx(3, 3)
        wait_idx(0, 0)
        start_g(0, 0)

        @pl.loop(0, nb // 4)
        def _(i):
            j = 4 * i
            for t in range(4):
                rj = t
                rn = (t + 1) % 4
                b = t % 2
                k_ = j + t

                @pl.when(k_ + 1 < nb)
                def _():
                    wait_idx(k_ + 1, rn)
                    start_g(rn, 1 - b)

                wait_g(rj, b)
                scat(rj, b)

                @pl.when(k_ + 4 < nb)
                def _():
                    start_idx(k_ + 4, rj)

        plsc.subcore_barrier()

        @pl.when(s < NS - 1)
        def _():
            pltpu.sync_copy(
                acc.at[pl.ds(base, rps0)],
                out_hbm.at[c].at[pl.ds(base, rps0)],
            )

        @pl.when(s == NS - 1)
        def _():
            pltpu.sync_copy(
                acc.at[pl.ds(base, rps_last)],
                out_hbm.at[c].at[pl.ds(base, rps_last)],
            )

    return k(g, idx)


_MT = 1000


def _tc_mlp1(x, W1, b1, Wc1):
    n, f = x.shape
    h = W1.shape[1]

    def body(x_ref, w1_ref, b1_ref, wc1_ref, out_ref):
        h1 = jnp.maximum(
            jnp.dot(x_ref[...], w1_ref[...], preferred_element_type=jnp.float32)
            + b1_ref[...],
            0.0,
        )
        out_ref[...] = jnp.dot(h1, wc1_ref[...], preferred_element_type=jnp.float32)

    return pl.pallas_call(
        body,
        grid=(n // _MT,),
        in_specs=[
            pl.BlockSpec((_MT, f), lambda i: (i, 0)),
            pl.BlockSpec((f, h), lambda i: (0, 0)),
            pl.BlockSpec((1, h), lambda i: (0, 0)),
            pl.BlockSpec((h, h), lambda i: (0, 0)),
        ],
        out_specs=pl.BlockSpec((_MT, h), lambda i: (i, 0)),
        out_shape=jax.ShapeDtypeStruct((n, h), jnp.float32),
    )(x, W1, b1.reshape(1, -1), Wc1)


def _tc_scale_split(partials, hw, fh):
    nw, n, _ = partials.shape
    h = hw.shape[1]

    def body(p_ref, hw_ref, g_ref, dis_ref):
        deg = jnp.sum(p_ref[...], axis=0) + 1.0
        dis = lax.rsqrt(deg)
        dis_ref[...] = dis
        g = hw_ref[...] * dis
        g_ref[0, :, :] = g[:, :fh]
        g_ref[1, :, :] = g[:, fh:]

    return pl.pallas_call(
        body,
        grid=(n // _MT,),
        in_specs=[
            pl.BlockSpec((nw, _MT, 1), lambda i: (0, i, 0)),
            pl.BlockSpec((_MT, h), lambda i: (i, 0)),
        ],
        out_specs=[
            pl.BlockSpec((NC, _MT, fh), lambda i: (0, i, 0)),
            pl.BlockSpec((_MT, 1), lambda i: (i, 0)),
        ],
        out_shape=[
            jax.ShapeDtypeStruct((NC, n, fh), jnp.float32),
            jax.ShapeDtypeStruct((n, 1), jnp.float32),
        ],
    )(partials, hw)


def _tc_mid(acc, dis, bc, Wc, fh):
    n = dis.shape[0]
    h = Wc.shape[0]

    def body(acc_ref, dis_ref, b_ref, w_ref, g_ref):
        a = jnp.concatenate([acc_ref[0], acc_ref[1]], axis=1)
        d = dis_ref[...]
        out1 = a * d + b_ref[...]
        h2 = jnp.where(out1 > 0, out1, 0.01 * out1)
        hw = jnp.dot(h2, w_ref[...], preferred_element_type=jnp.float32)
        g = hw * d
        g_ref[0, :, :] = g[:, :fh]
        g_ref[1, :, :] = g[:, fh:]

    return pl.pallas_call(
        body,
        grid=(n // _MT,),
        in_specs=[
            pl.BlockSpec((NC, _MT, fh), lambda i: (0, i, 0)),
            pl.BlockSpec((_MT, 1), lambda i: (i, 0)),
            pl.BlockSpec((1, h), lambda i: (0, 0)),
            pl.BlockSpec((h, h), lambda i: (0, 0)),
        ],
        out_specs=pl.BlockSpec((NC, _MT, fh), lambda i: (0, i, 0)),
        out_shape=jax.ShapeDtypeStruct((NC, n, fh), jnp.float32),
    )(acc, dis, bc.reshape(1, -1), Wc)


def _tc_final(acc, dis, bc, W2, b2, batch2d, n_graphs, fh):
    n = dis.shape[0]
    h = W2.shape[0]

    def body(acc_ref, dis_ref, bc_ref, w_ref, b2_ref, batch_ref, emb_ref, h4_ref):
        i = pl.program_id(0)
        a = jnp.concatenate([acc_ref[0], acc_ref[1]], axis=1)
        out2 = a * dis_ref[...] + bc_ref[...]
        h3 = jnp.where(out2 > 0, out2, 0.01 * out2)
        h4 = jnp.dot(h3, w_ref[...], preferred_element_type=jnp.float32) + b2_ref[...]
        h4_ref[...] = h4
        seg = batch_ref[...]
        ids = lax.broadcasted_iota(jnp.int32, (_MT, n_graphs), 1)
        oh = (seg == ids).astype(jnp.float32)
        part = lax.dot_general(
            oh, h4, (((0,), (0,)), ((), ())),
            preferred_element_type=jnp.float32,
        )

        @pl.when(i == 0)
        def _():
            emb_ref[...] = part

        @pl.when(i > 0)
        def _():
            emb_ref[...] += part

    return pl.pallas_call(
        body,
        grid=(n // _MT,),
        in_specs=[
            pl.BlockSpec((NC, _MT, fh), lambda i: (0, i, 0)),
            pl.BlockSpec((_MT, 1), lambda i: (i, 0)),
            pl.BlockSpec((1, h), lambda i: (0, 0)),
            pl.BlockSpec((h, h), lambda i: (0, 0)),
            pl.BlockSpec((1, h), lambda i: (0, 0)),
            pl.BlockSpec((_MT, 1), lambda i: (i, 0)),
        ],
        out_specs=[
            pl.BlockSpec((n_graphs, h), lambda i: (0, 0)),
            pl.BlockSpec((_MT, h), lambda i: (i, 0)),
        ],
        out_shape=[
            jax.ShapeDtypeStruct((n_graphs, h), jnp.float32),
            jax.ShapeDtypeStruct((n, h), jnp.float32),
        ],
    )(acc, dis, bc.reshape(1, -1), W2, b2.reshape(1, -1), batch2d)


def kernel(x, edge_index, batch, W1, b1, Wc1, bc1, Wc2, bc2, W2, b2):
    n, _ = x.shape
    h = W1.shape[1]
    fh = h // NC
    e = edge_index.shape[1]
    g_count = 64

    src = edge_index[0]
    dst = edge_index[1]

    eps_real = e // NS
    nb = (-(-eps_real // CH) + 3) // 4 * 4
    epw = nb * CH
    padk = epw - eps_real
    src_r = src.reshape(NS, eps_real)
    dst_r = dst.reshape(NS, eps_real)
    if padk:
        pad_src = (
            jnp.arange(padk, dtype=jnp.int32)[None, :] * 131
            + jnp.arange(NS, dtype=jnp.int32)[:, None] * 977
        ) % n
        pad_dst = jnp.broadcast_to(
            n + (jnp.arange(padk, dtype=jnp.int32) % TRASH)[None, :], (NS, padk)
        )
        src_p = jnp.concatenate([src_r, pad_src], axis=1)
        dst_p = jnp.concatenate([dst_r, pad_dst], axis=1)
    else:
        src_p, dst_p = src_r, dst_r
    dst_c = dst_p.reshape(NS, nb, 1, CH)
    idx = jnp.stack(
        [
            jnp.concatenate([src_p.reshape(NS, nb, 1, CH), dst_c], axis=2),
            jnp.concatenate([(src_p + n).reshape(NS, nb, 1, CH), dst_c], axis=2),
        ]
    )

    partials = _sc_degree(dst, n).reshape(NC * NS, n, 1)
    hw1 = _tc_mlp1(x, W1, b1, Wc1)
    g1, dis = _tc_scale_split(partials, hw1, fh)
    acc1 = _sc_scatter(g1.reshape(NC * n, fh), idx, n, fh)
    g2 = _tc_mid(acc1, dis, bc1, Wc2, fh)
    acc2 = _sc_scatter(g2.reshape(NC * n, fh), idx, n, fh)
    emb, h4 = _tc_final(acc2, dis, bc2, W2, b2, batch.reshape(n, 1), g_count, fh)
    return emb, h4
